# baseline (speedup 1.0000x reference)
import jax
import jax.numpy as jnp
from jax import lax
from jax.experimental import pallas as pl
from jax.experimental.pallas import tpu as pltpu
from jax.experimental.pallas import tpu_sc as plsc

BUFFER_SIZE = 65536
VALUE_DIM = 256
BATCH = 16384

_INFO = plsc.get_sparse_core_info()
NUM_CORES = _INFO.num_cores
NUM_SUBCORES = _INFO.num_subcores
LANES = _INFO.num_lanes
SCRANGE = BUFFER_SIZE // NUM_CORES
RBITS = 15
STRIPE = SCRANGE // NUM_SUBCORES
MSUB = 512
SLICE = BATCH // NUM_SUBCORES
SVECS = SLICE // LANES
UNROLL = 4
CH = 16
CAPQ = SLICE + LANES
CAPF = SLICE + 2 * CH
NBUF = 8
PF = 6


def _body(mem_hbm, idx_hbm, val_hbm, off_hbm, out_hbm,
          table, tmerge, parts, idx_buf, off_buf, qenc, list_i, list_x,
          r0, r1, r2, r3, r4, r5, r6, r7,
          g0, g1, g2, g3, g4, g5, g6, g7,
          s0, s1, s2, s3, s4, s5, s6, s7,
          sem_i, sem_o, sem_m):
  rows = [r0, r1, r2, r3, r4, r5, r6, r7]
  semg = [g0, g1, g2, g3, g4, g5, g6, g7]
  sems = [s0, s1, s2, s3, s4, s5, s6, s7]
  cid = lax.axis_index("c")
  sid = lax.axis_index("s")
  lo = cid * SCRANGE
  hi = lo + SCRANGE
  base = sid * SLICE
  iota = lax.iota(jnp.int32, LANES)

  cp_i = pltpu.async_copy(idx_hbm.at[pl.ds(base, SLICE)], idx_buf, sem_i)
  cp_o = pltpu.async_copy(off_hbm.at[pl.ds(base, SLICE)], off_buf, sem_o)

  neg1 = jnp.full((LANES,), -1, jnp.int32)

  def init_body(k, _):
    for u in range(8):
      table[pl.ds((k * 8 + u) * LANES, LANES)] = neg1
    return 0

  lax.fori_loop(0, SCRANGE // LANES // 8, init_body, 0)
  cp_i.wait()
  cp_o.wait()

  def scan_body(k0, carry):
    acc, pq = carry
    ivs, ovs = [], []
    for u in range(UNROLL):
      k = k0 * UNROLL + u
      ivs.append(idx_buf[pl.ds(k * LANES, LANES)])
      ovs.append(off_buf[pl.ds(k * LANES, LANES)])
    lis, jvs, ms = [], [], []
    for u in range(UNROLL):
      k = k0 * UNROLL + u
      iv, ov = ivs[u], ovs[u]
      jv = iota + (base + k * LANES)
      m = (iv >= lo) & (iv < hi)
      li = iv - lo
      plsc.store_scatter(table, [li], jv, mask=m)
      lis.append(li); jvs.append(jv); ms.append(m)

      mo = (ov >= lo) & (ov < hi)
      mo32 = mo.astype(jnp.int32)
      q = pq + jnp.cumsum(mo32) - mo32
      enc = (jv << RBITS) | (ov - lo)
      plsc.store_scatter(qenc, [q], enc, mask=mo)
      pq = pq + plsc.all_reduce_population_count(mo)
    for u in range(UNROLL):
      w = plsc.load_gather(table, [lis[u]], mask=ms[u])
      acc = acc | (ms[u] & (w < jvs[u]))
    return acc, pq

  false16 = jnp.zeros((LANES,), jnp.bool_)
  zero = jnp.zeros((LANES,), jnp.int32)
  lost, pq = lax.fori_loop(0, SVECS // UNROLL, scan_body, (false16, zero))
  n_in = jnp.max(pq)

  def fix_pass(anyw):
    def body(k, acc):
      iv = idx_buf[pl.ds(k * LANES, LANES)]
      jv = iota + (base + k * LANES)
      m = (iv >= lo) & (iv < hi)
      li = iv - lo
      w = plsc.load_gather(table, [li], mask=m)
      wrong = m & (w < jv)
      plsc.store_scatter(table, [li], jv, mask=wrong)
      return acc | wrong

    acc = lax.fori_loop(0, SVECS, body, false16)
    return jnp.any(acc)

  lax.while_loop(lambda s: s, fix_pass, jnp.any(lost))

  pltpu.sync_copy(table, parts.at[sid])
  plsc.subcore_barrier()

  for s in range(STRIPE // MSUB):
    sub = pl.ds(sid * STRIPE + s * MSUB, MSUB)
    for r in range(NUM_SUBCORES):
      pltpu.async_copy(parts.at[r, sub], tmerge.at[r], sem_m)
    for r in range(NUM_SUBCORES):
      pltpu.make_async_copy(parts.at[r, sub], tmerge.at[r], sem_m).wait()

    def merge_body(v, _):
      sl = pl.ds(v * LANES, LANES)
      acc = tmerge[0, sl]
      for r in range(1, NUM_SUBCORES):
        p = tmerge[r, sl]
        acc = jnp.where(p >= 0, p, acc)
      tmerge[0, sl] = acc
      return 0

    lax.fori_loop(0, MSUB // LANES, merge_body, 0)
    pltpu.sync_copy(tmerge.at[0], parts.at[0, sub])
  plsc.subcore_barrier()
  pltpu.sync_copy(parts.at[0], table)

  @pl.when(n_in % LANES != 0)
  def _():
    lastq = jnp.full((LANES,), n_in - 1, jnp.int32)
    le = plsc.load_gather(qenc, [lastq])
    plsc.store_scatter(qenc, [n_in + iota], le)

  n_inr = ((n_in + LANES - 1) // LANES) * LANES

  def split_body(k, carry):
    ph, pm = carry
    enc = qenc[pl.ds(k * LANES, LANES)]
    pos = enc >> RBITS
    li = enc & (SCRANGE - 1)
    r = plsc.load_gather(table, [li])
    hit = r >= 0
    miss = ~hit
    h32 = hit.astype(jnp.int32)
    m32 = miss.astype(jnp.int32)
    hq = ph + jnp.cumsum(h32) - h32
    mq = (CAPF - 1) - (pm + jnp.cumsum(m32) - m32)
    plsc.store_scatter(list_i, [hq], pos, mask=hit)
    plsc.store_scatter(list_x, [hq], r, mask=hit)
    plsc.store_scatter(list_i, [mq], pos, mask=miss)
    plsc.store_scatter(list_x, [mq], li + lo, mask=miss)
    ph = ph + plsc.all_reduce_population_count(hit)
    pm = pm + plsc.all_reduce_population_count(miss)
    return ph, pm

  ph, pm = lax.fori_loop(0, n_inr // LANES, split_body, (zero, zero))
  n_hit = jnp.max(ph)
  n_miss = jnp.max(pm)

  def pad(n, flat_of):
    @pl.when(n % CH != 0)
    def _():
      lastq = flat_of(jnp.full((LANES,), n - 1, jnp.int32))
      li_ = plsc.load_gather(list_i, [lastq])
      lx_ = plsc.load_gather(list_x, [lastq])
      for u in range(CH // LANES):
        tail = flat_of(n + u * LANES + iota)
        plsc.store_scatter(list_i, [tail], li_)
        plsc.store_scatter(list_x, [tail], lx_)

  pad(n_hit, lambda t: t)
  pad(n_miss, lambda t: (CAPF - 1) - t)

  def move(src_hbm, n, start_of):
    nch = (n + CH - 1) // CH

    def xs(c):
      return list_x.at[pl.ds(start_of(c), CH)]

    def js(c):
      return list_i.at[pl.ds(start_of(c), CH)]

    for b in range(PF):
      @pl.when(b < nch)
      def _(b=b):
        pltpu.async_copy(src_hbm.at[xs(b)], rows[b], semg[b])

    def outer(t, _):
      c0 = t * NBUF
      for b in range(NBUF):
        c = c0 + b

        @pl.when(c < nch)
        def _(b=b, c=c):
          pltpu.make_async_copy(src_hbm.at[xs(c)], rows[b], semg[b]).wait()
          pltpu.async_copy(rows[b], out_hbm.at[js(c)], sems[b])
          @pl.when(c + PF < nch)
          def _():
            b2 = (b + PF) % NBUF

            @pl.when(c >= NBUF - PF)
            def _():
              pltpu.make_async_copy(rows[b2], out_hbm.at[js(0)],
                                    sems[b2]).wait()
            pltpu.async_copy(src_hbm.at[xs(c + PF)], rows[b2], semg[b2])
      return 0

    lax.fori_loop(0, (nch + NBUF - 1) // NBUF, outer, 0)

    for b in range(NBUF):
      @pl.when(b < nch)
      def _(b=b):
        pltpu.make_async_copy(rows[b], out_hbm.at[js(0)], sems[b]).wait()

  move(val_hbm, n_hit, lambda c: c * CH)
  move(mem_hbm, n_miss, lambda c: CAPF - (c + 1) * CH)


@jax.jit
def kernel(mem, idx, val, offset):
  mesh = plsc.VectorSubcoreMesh(core_axis_name="c", subcore_axis_name="s")
  fn = pl.kernel(
      _body,
      out_type=jax.ShapeDtypeStruct((BATCH, VALUE_DIM), jnp.float32),
      mesh=mesh,
      scratch_types=(
          [
              pltpu.VMEM((SCRANGE,), jnp.int32),
              pltpu.VMEM((NUM_SUBCORES, MSUB), jnp.int32),
              pltpu.VMEM_SHARED((NUM_SUBCORES, SCRANGE), jnp.int32),
              pltpu.VMEM((SLICE,), jnp.int32),
              pltpu.VMEM((SLICE,), jnp.int32),
              pltpu.VMEM((CAPQ,), jnp.int32),
              pltpu.VMEM((CAPF,), jnp.int32),
              pltpu.VMEM((CAPF,), jnp.int32),
          ]
          + [pltpu.VMEM((CH, VALUE_DIM), jnp.float32) for _ in range(NBUF)]
          + [pltpu.SemaphoreType.DMA for _ in range(2 * NBUF + 3)]
      ),
      compiler_params=pltpu.CompilerParams(needs_layout_passes=False),
  )
  return fn(mem, idx.astype(jnp.int32), val, offset.astype(jnp.int32))


# speedup vs baseline: 1.0181x; 1.0181x over previous
import jax
import jax.numpy as jnp
from jax import lax
from jax.experimental import pallas as pl
from jax.experimental.pallas import tpu as pltpu
from jax.experimental.pallas import tpu_sc as plsc

BUFFER_SIZE = 65536
VALUE_DIM = 256
BATCH = 16384

_INFO = plsc.get_sparse_core_info()
NUM_CORES = _INFO.num_cores
NUM_SUBCORES = _INFO.num_subcores
NUM_TILES = NUM_CORES * NUM_SUBCORES
LANES = _INFO.num_lanes
RANGE = BUFFER_SIZE // NUM_TILES
RBITS = 11
NVECS = BATCH // LANES
UNROLL = 4
CH = 16
CAPQ = BATCH + LANES
CAPF = BATCH + 2 * CH
NBUF = 10
PF = 8


def _body(mem_hbm, idx_hbm, val_hbm, off_hbm, out_hbm,
          table, idx_buf, off_buf, qenc, list_i, list_x,
          r0, r1, r2, r3, r4, r5, r6, r7, r8, r9,
          g0, g1, g2, g3, g4, g5, g6, g7, g8, g9,
          s0, s1, s2, s3, s4, s5, s6, s7, s8, s9,
          sem_i, sem_o):
  rows = [r0, r1, r2, r3, r4, r5, r6, r7, r8, r9]
  semg = [g0, g1, g2, g3, g4, g5, g6, g7, g8, g9]
  sems = [s0, s1, s2, s3, s4, s5, s6, s7, s8, s9]
  wid = lax.axis_index("s") * NUM_CORES + lax.axis_index("c")
  lo = wid * RANGE
  hi = lo + RANGE
  iota = lax.iota(jnp.int32, LANES)

  cp_i = pltpu.async_copy(idx_hbm, idx_buf, sem_i)
  cp_o = pltpu.async_copy(off_hbm, off_buf, sem_o)

  neg1 = jnp.full((LANES,), -1, jnp.int32)

  def init_body(k, _):
    for u in range(4):
      table[pl.ds((k * 4 + u) * LANES, LANES)] = neg1
    return 0

  lax.fori_loop(0, RANGE // LANES // 4, init_body, 0)
  cp_i.wait()
  cp_o.wait()

  def scan_body(k0, carry):
    acc, pq = carry
    ivs, ovs = [], []
    for u in range(UNROLL):
      k = k0 * UNROLL + u
      ivs.append(idx_buf[pl.ds(k * LANES, LANES)])
      ovs.append(off_buf[pl.ds(k * LANES, LANES)])
    lis, jvs, ms = [], [], []
    for u in range(UNROLL):
      k = k0 * UNROLL + u
      iv, ov = ivs[u], ovs[u]
      jv = iota + k * LANES
      m = (iv >= lo) & (iv < hi)
      li = iv - lo
      plsc.store_scatter(table, [li], jv, mask=m)
      lis.append(li); jvs.append(jv); ms.append(m)

      mo = (ov >= lo) & (ov < hi)
      mo32 = mo.astype(jnp.int32)
      q = pq + jnp.cumsum(mo32) - mo32
      enc = (jv << RBITS) | (ov - lo)
      plsc.store_scatter(qenc, [q], enc, mask=mo)
      pq = pq + plsc.all_reduce_population_count(mo)
    for u in range(UNROLL):
      w = plsc.load_gather(table, [lis[u]], mask=ms[u])
      acc = acc | (ms[u] & (w < jvs[u]))
    return acc, pq

  false16 = jnp.zeros((LANES,), jnp.bool_)
  zero = jnp.zeros((LANES,), jnp.int32)
  lost, pq = lax.fori_loop(0, NVECS // UNROLL, scan_body, (false16, zero))
  n_in = jnp.max(pq)

  def fix_pass(anyw):
    def body(k, acc):
      iv = idx_buf[pl.ds(k * LANES, LANES)]
      jv = iota + k * LANES
      m = (iv >= lo) & (iv < hi)
      li = iv - lo
      w = plsc.load_gather(table, [li], mask=m)
      wrong = m & (w < jv)
      plsc.store_scatter(table, [li], jv, mask=wrong)
      return acc | wrong

    acc = lax.fori_loop(0, NVECS, body, false16)
    return jnp.any(acc)

  lax.while_loop(lambda s: s, fix_pass, jnp.any(lost))

  @pl.when(n_in % LANES != 0)
  def _():
    lastq = jnp.full((LANES,), n_in - 1, jnp.int32)
    le = plsc.load_gather(qenc, [lastq])
    plsc.store_scatter(qenc, [n_in + iota], le)

  n_inr = ((n_in + LANES - 1) // LANES) * LANES

  def split_body(k, carry):
    ph, pm = carry
    enc = qenc[pl.ds(k * LANES, LANES)]
    pos = enc >> RBITS
    li = enc & (RANGE - 1)
    r = plsc.load_gather(table, [li])
    hit = r >= 0
    miss = ~hit
    h32 = hit.astype(jnp.int32)
    m32 = miss.astype(jnp.int32)
    hq = ph + jnp.cumsum(h32) - h32
    mq = (CAPF - 1) - (pm + jnp.cumsum(m32) - m32)
    plsc.store_scatter(list_i, [hq], pos, mask=hit)
    plsc.store_scatter(list_x, [hq], r, mask=hit)
    plsc.store_scatter(list_i, [mq], pos, mask=miss)
    plsc.store_scatter(list_x, [mq], li + lo, mask=miss)
    ph = ph + plsc.all_reduce_population_count(hit)
    pm = pm + plsc.all_reduce_population_count(miss)
    return ph, pm

  ph, pm = lax.fori_loop(0, n_inr // LANES, split_body, (zero, zero))
  n_hit = jnp.max(ph)
  n_miss = jnp.max(pm)

  def pad(n, flat_of):
    @pl.when(n % CH != 0)
    def _():
      lastq = flat_of(jnp.full((LANES,), n - 1, jnp.int32))
      li_ = plsc.load_gather(list_i, [lastq])
      lx_ = plsc.load_gather(list_x, [lastq])
      for u in range(CH // LANES):
        tail = flat_of(n + u * LANES + iota)
        plsc.store_scatter(list_i, [tail], li_)
        plsc.store_scatter(list_x, [tail], lx_)

  pad(n_hit, lambda t: t)
  pad(n_miss, lambda t: (CAPF - 1) - t)

  def move(src_hbm, n, start_of):
    nch = (n + CH - 1) // CH

    def xs(c):
      return list_x.at[pl.ds(start_of(c), CH)]

    def js(c):
      return list_i.at[pl.ds(start_of(c), CH)]

    for b in range(PF):
      @pl.when(b < nch)
      def _(b=b):
        pltpu.async_copy(src_hbm.at[xs(b)], rows[b], semg[b])

    def outer(t, _):
      c0 = t * NBUF
      for b in range(NBUF):
        c = c0 + b

        @pl.when(c < nch)
        def _(b=b, c=c):
          pltpu.make_async_copy(src_hbm.at[xs(c)], rows[b], semg[b]).wait()
          pltpu.async_copy(rows[b], out_hbm.at[js(c)], sems[b])
          @pl.when(c + PF < nch)
          def _():
            b2 = (b + PF) % NBUF

            @pl.when(c >= NBUF - PF)
            def _():
              pltpu.make_async_copy(rows[b2], out_hbm.at[js(0)],
                                    sems[b2]).wait()
            pltpu.async_copy(src_hbm.at[xs(c + PF)], rows[b2], semg[b2])
      return 0

    lax.fori_loop(0, (nch + NBUF - 1) // NBUF, outer, 0)

    for b in range(NBUF):
      @pl.when(b < nch)
      def _(b=b):
        pltpu.make_async_copy(rows[b], out_hbm.at[js(0)], sems[b]).wait()

  move(val_hbm, n_hit, lambda c: c * CH)
  move(mem_hbm, n_miss, lambda c: CAPF - (c + 1) * CH)


@jax.jit
def kernel(mem, idx, val, offset):
  mesh = plsc.VectorSubcoreMesh(core_axis_name="c", subcore_axis_name="s")
  fn = pl.kernel(
      _body,
      out_type=jax.ShapeDtypeStruct((BATCH, VALUE_DIM), jnp.float32),
      mesh=mesh,
      scratch_types=(
          [
              pltpu.VMEM((RANGE,), jnp.int32),
              pltpu.VMEM((BATCH,), jnp.int32),
              pltpu.VMEM((BATCH,), jnp.int32),
              pltpu.VMEM((CAPQ,), jnp.int32),
              pltpu.VMEM((CAPF,), jnp.int32),
              pltpu.VMEM((CAPF,), jnp.int32),
          ]
          + [pltpu.VMEM((CH, VALUE_DIM), jnp.float32) for _ in range(NBUF)]
          + [pltpu.SemaphoreType.DMA for _ in range(2 * NBUF + 2)]
      ),
      compiler_params=pltpu.CompilerParams(needs_layout_passes=False),
  )
  return fn(mem, idx.astype(jnp.int32), val, offset.astype(jnp.int32))
